# baseline (speedup 1.0000x reference)
import functools

import jax
import jax.numpy as jnp
from jax import lax
from jax.experimental import pallas as pl
from jax.experimental.pallas import tpu as pltpu
from jax.experimental.pallas import tpu_sc as plsc

K = 16
W1 = 10
W2 = 20
NC = 2
NS = 16
NW = NC * NS


def _mlp_body(x_ref, w0_ref, b0_ref, w1_ref, b1_ref, o_ref):
    x = x_ref[0]
    h = jnp.maximum(
        jnp.dot(x, w0_ref[...], preferred_element_type=jnp.float32) + b0_ref[...], 0.0)
    o = jnp.maximum(
        jnp.dot(h, w1_ref[...], preferred_element_type=jnp.float32) + b1_ref[...], 0.0)
    o_ref[0] = o


def _mlp(x, w0, b0, w1, b1):
    b, n, cin = x.shape
    h = w0.shape[0]
    cout = w1.shape[0]
    return pl.pallas_call(
        _mlp_body,
        grid=(b,),
        in_specs=[
            pl.BlockSpec((1, n, cin), lambda i: (i, 0, 0)),
            pl.BlockSpec((cin, h), lambda i: (0, 0)),
            pl.BlockSpec((1, h), lambda i: (0, 0)),
            pl.BlockSpec((h, cout), lambda i: (0, 0)),
            pl.BlockSpec((1, cout), lambda i: (0, 0)),
        ],
        out_specs=pl.BlockSpec((1, n, cout), lambda i: (i, 0, 0)),
        out_shape=jax.ShapeDtypeStruct((b, n, cout), jnp.float32),
    )(x, w0.T, b0.reshape(1, h), w1.T, b1.reshape(1, cout))


def _topk_body(qx_ref, rx_ref, o_ref, s_scr, n):
    bidx = pl.program_id(0)
    q = qx_ref[0]
    r3 = rx_ref[0]
    rsq = jnp.sum(r3 * r3, axis=0, keepdims=True)
    qsq = jnp.sum(q * q, axis=1, keepdims=True)
    s_scr[...] = (qsq + rsq) - 2.0 * jnp.dot(q, r3,
                                             preferred_element_type=jnp.float32)
    lane = jax.lax.broadcasted_iota(jnp.int32, s_scr.shape, 1)
    base = bidx * n
    for t in range(K):
        s = s_scr[...]
        am = jnp.argmin(s, axis=1, keepdims=True).astype(jnp.int32)
        s_scr[...] = jnp.where(lane == am, jnp.inf, s)
        o_ref[0, :, t:t + 1] = am + base


def _topk(rxyz, qxyz_t, bm):
    b, _, n = rxyz.shape
    m = qxyz_t.shape[1]
    return pl.pallas_call(
        functools.partial(_topk_body, n=n),
        grid=(b, m // bm),
        in_specs=[
            pl.BlockSpec((1, bm, 3), lambda i, j: (i, j, 0)),
            pl.BlockSpec((1, 3, n), lambda i, j: (i, 0, 0)),
        ],
        out_specs=pl.BlockSpec((1, bm, K), lambda i, j: (i, j, 0)),
        out_shape=jax.ShapeDtypeStruct((b, m, K), jnp.int32),
        scratch_shapes=[pltpu.VMEM((bm, n), jnp.float32)],
    )(qxyz_t, rxyz)


def _sc_gather(table_flat, idx_flat):
    g_total = idx_flat.shape[0]
    d = table_flat.shape[1]
    gpw = g_total // NW
    csz = min(128, gpw)
    nchunk = gpw // csz
    mesh = plsc.VectorSubcoreMesh(core_axis_name="c", subcore_axis_name="s")

    @functools.partial(
        pl.kernel,
        out_type=jax.ShapeDtypeStruct((g_total, d), jnp.float32),
        mesh=mesh,
        scratch_types=[
            pltpu.VMEM((gpw,), jnp.int32),
            pltpu.VMEM((csz, d), jnp.float32),
            pltpu.VMEM((csz, d), jnp.float32),
            pltpu.SemaphoreType.DMA,
            pltpu.SemaphoreType.DMA,
        ],
    )
    def gk(idx_hbm, tab_hbm, out_hbm, idx_v, buf0, buf1, semin, semout):
        wid = lax.axis_index("s") * NC + lax.axis_index("c")
        base = wid * gpw
        bufs = [buf0, buf1]
        pltpu.sync_copy(idx_hbm.at[pl.ds(base, gpw)], idx_v)
        hin = [None] * nchunk
        hout = [None] * nchunk
        hin[0] = pltpu.async_copy(
            tab_hbm.at[idx_v.at[pl.ds(0, csz)]], bufs[0], semin)
        for j in range(nchunk):
            if j + 1 < nchunk:
                if j >= 1:
                    hout[j - 1].wait()
                hin[j + 1] = pltpu.async_copy(
                    tab_hbm.at[idx_v.at[pl.ds((j + 1) * csz, csz)]],
                    bufs[(j + 1) % 2], semin)
            hin[j].wait()
            hout[j] = pltpu.async_copy(
                bufs[j % 2], out_hbm.at[pl.ds(base + j * csz, csz)], semout)
        hout[nchunk - 1].wait()
        if nchunk >= 2:
            hout[nchunk - 2].wait()

    return gk(idx_flat, table_flat)


def _stagec_body(g_ref, qx_ref, wn0_ref, wb0_ref, wn1_ref, wb1_ref, lw_ref,
                 lb_ref, o_ref, c, dpad):
    g = g_ref[0]
    q = qx_ref[0]
    bmq = q.shape[0]
    mk = bmq * K

    ri = jax.lax.broadcasted_iota(jnp.int32, (mk, bmq), 0)
    ci = jax.lax.broadcasted_iota(jnp.int32, (mk, bmq), 1)
    rep = (ri // K == ci).astype(jnp.float32)
    q_rep = jnp.dot(rep, q, preferred_element_type=jnp.float32, precision=jax.lax.Precision.HIGHEST)

    rel = g[:, :3] - q_rep
    h1 = jnp.maximum(
        jnp.dot(rel, wn0_ref[...], preferred_element_type=jnp.float32)
        + wb0_ref[...], 0.0)
    w = jnp.maximum(
        jnp.dot(h1, wn1_ref[...], preferred_element_type=jnp.float32)
        + wb1_ref[...], 0.0)
    f = g[:, 3:3 + c]

    fcol = jax.lax.broadcasted_iota(jnp.int32, (c, W2 * c), 1)
    frow = jax.lax.broadcasted_iota(jnp.int32, (c, W2 * c), 0)
    fmat = (fcol - (fcol // c) * c == frow).astype(jnp.float32)
    ecol = jax.lax.broadcasted_iota(jnp.int32, (W2, W2 * c), 1)
    erow = jax.lax.broadcasted_iota(jnp.int32, (W2, W2 * c), 0)
    emat = (ecol // c == erow).astype(jnp.float32)

    p = (jnp.dot(f, fmat, preferred_element_type=jnp.float32, precision=jax.lax.Precision.HIGHEST)
         * jnp.dot(w, emat, preferred_element_type=jnp.float32, precision=jax.lax.Precision.HIGHEST))

    rti = jax.lax.broadcasted_iota(jnp.int32, (bmq, mk), 0)
    cti = jax.lax.broadcasted_iota(jnp.int32, (bmq, mk), 1)
    red = (cti // K == rti).astype(jnp.float32)
    z = jnp.dot(red, p, preferred_element_type=jnp.float32, precision=jax.lax.Precision.HIGHEST)

    out = jnp.dot(z, lw_ref[...], preferred_element_type=jnp.float32) + lb_ref[...]
    o_ref[0] = jnp.maximum(out, 0.0)


def _stagec(g, qxyz_t, wn_w0, wn_b0, wn_w1, wn_b1, lin_w, lin_b, c, dpad, bmq):
    b, m, _ = qxyz_t.shape
    lw_perm = lin_w.reshape(c, c, W2).transpose(0, 2, 1).reshape(c, W2 * c).T
    return pl.pallas_call(
        functools.partial(_stagec_body, c=c, dpad=dpad),
        grid=(b, m // bmq),
        in_specs=[
            pl.BlockSpec((1, bmq * K, dpad), lambda i, j: (i, j, 0)),
            pl.BlockSpec((1, bmq, 3), lambda i, j: (i, j, 0)),
            pl.BlockSpec((3, W1), lambda i, j: (0, 0)),
            pl.BlockSpec((1, W1), lambda i, j: (0, 0)),
            pl.BlockSpec((W1, W2), lambda i, j: (0, 0)),
            pl.BlockSpec((1, W2), lambda i, j: (0, 0)),
            pl.BlockSpec((W2 * c, c), lambda i, j: (0, 0)),
            pl.BlockSpec((1, c), lambda i, j: (0, 0)),
        ],
        out_specs=pl.BlockSpec((1, bmq, c), lambda i, j: (i, j, 0)),
        out_shape=jax.ShapeDtypeStruct((b, m, c), jnp.float32),
    )(g, qxyz_t, wn_w0.T, wn_b0.reshape(1, W1), wn_w1.T,
      wn_b1.reshape(1, W2), lw_perm, lin_b.reshape(1, -1))


def kernel(xyz0, xyz1, xyz2, xyz3,
           mlp0_w0, mlp0_b0, mlp0_w1, mlp0_b1,
           mlp1_w0, mlp1_b0, mlp1_w1, mlp1_b1,
           mlp2_w0, mlp2_b0, mlp2_w1, mlp2_b1,
           conv0_wn_w0, conv0_wn_b0, conv0_wn_w1, conv0_wn_b1, conv0_lin_w, conv0_lin_b,
           conv1_wn_w0, conv1_wn_b0, conv1_wn_w1, conv1_wn_b1, conv1_lin_w, conv1_lin_b,
           conv2_wn_w0, conv2_wn_b0, conv2_wn_w1, conv2_wn_b1, conv2_lin_w, conv2_lin_b):
    xyzs = [xyz0, xyz1, xyz2, xyz3]
    mlps = [(mlp0_w0, mlp0_b0, mlp0_w1, mlp0_b1),
            (mlp1_w0, mlp1_b0, mlp1_w1, mlp1_b1),
            (mlp2_w0, mlp2_b0, mlp2_w1, mlp2_b1)]
    convs = [(conv0_wn_w0, conv0_wn_b0, conv0_wn_w1, conv0_wn_b1, conv0_lin_w, conv0_lin_b),
             (conv1_wn_w0, conv1_wn_b0, conv1_wn_w1, conv1_wn_b1, conv1_lin_w, conv1_lin_b),
             (conv2_wn_w0, conv2_wn_b0, conv2_wn_w1, conv2_wn_b1, conv2_lin_w, conv2_lin_b)]
    dpads = [128, 128, 256]
    bmqs = [128, 64, 32]

    idxs = [_topk(xyzs[i], xyzs[i + 1].transpose(0, 2, 1), 128)
            for i in range(3)]

    outs = []
    prev = xyz0.transpose(0, 2, 1)
    for i in range(3):
        b, n, _ = prev.shape
        m = xyzs[i + 1].shape[2]
        w0, b0, w1, b1 = mlps[i]
        feat_t = _mlp(prev, w0, b0, w1, b1)
        c = feat_t.shape[-1]
        dpad = dpads[i]
        tab = jnp.concatenate(
            [xyzs[i].transpose(0, 2, 1), feat_t,
             jnp.zeros((b, n, dpad - 3 - c), jnp.float32)], axis=-1)
        g = _sc_gather(tab.reshape(b * n, dpad), idxs[i].reshape(-1))
        cwn0, cb0, cwn1, cb1, lw, lb = convs[i]
        qxyz_t = xyzs[i + 1].transpose(0, 2, 1)
        out_t = _stagec(g.reshape(b, m * K, dpad), qxyz_t, cwn0, cb0, cwn1,
                        cb1, lw, lb, c, dpad, bmqs[i])
        outs.append(out_t.transpose(0, 2, 1))
        prev = out_t
    return tuple(outs)


# speedup vs baseline: 1.1726x; 1.1726x over previous
import functools

import jax
import jax.numpy as jnp
from jax import lax
from jax.experimental import pallas as pl
from jax.experimental.pallas import tpu as pltpu
from jax.experimental.pallas import tpu_sc as plsc

K = 16
W1 = 10
W2 = 20
NC = 2
NS = 16
NW = NC * NS


def _mlp_body(x_ref, w0_ref, b0_ref, w1_ref, b1_ref, o_ref):
    x = x_ref[0]
    h = jnp.maximum(
        jnp.dot(x, w0_ref[...], preferred_element_type=jnp.float32) + b0_ref[...], 0.0)
    o = jnp.maximum(
        jnp.dot(h, w1_ref[...], preferred_element_type=jnp.float32) + b1_ref[...], 0.0)
    o_ref[0] = o


def _mlp(x, w0, b0, w1, b1):
    b, n, cin = x.shape
    h = w0.shape[0]
    cout = w1.shape[0]
    return pl.pallas_call(
        _mlp_body,
        grid=(b,),
        in_specs=[
            pl.BlockSpec((1, n, cin), lambda i: (i, 0, 0)),
            pl.BlockSpec((cin, h), lambda i: (0, 0)),
            pl.BlockSpec((1, h), lambda i: (0, 0)),
            pl.BlockSpec((h, cout), lambda i: (0, 0)),
            pl.BlockSpec((1, cout), lambda i: (0, 0)),
        ],
        out_specs=pl.BlockSpec((1, n, cout), lambda i: (i, 0, 0)),
        out_shape=jax.ShapeDtypeStruct((b, n, cout), jnp.float32),
    )(x, w0.T, b0.reshape(1, h), w1.T, b1.reshape(1, cout))


def _topk_body(qx_ref, rx_ref, o_ref, s_scr, n):
    bidx = pl.program_id(0)
    q = qx_ref[0]
    r3 = rx_ref[0]
    rsq = jnp.sum(r3 * r3, axis=0, keepdims=True)
    qsq = jnp.sum(q * q, axis=1, keepdims=True)
    s_scr[...] = (qsq + rsq) - 2.0 * jnp.dot(q, r3,
                                             preferred_element_type=jnp.float32)
    lane = jax.lax.broadcasted_iota(jnp.int32, s_scr.shape, 1)
    base = bidx * n
    for t in range(K):
        s = s_scr[...]
        am = jnp.argmin(s, axis=1, keepdims=True).astype(jnp.int32)
        s_scr[...] = jnp.where(lane == am, jnp.inf, s)
        o_ref[0, :, t:t + 1] = am + base


def _topk(rxyz, qxyz_t, bm):
    b, _, n = rxyz.shape
    m = qxyz_t.shape[1]
    return pl.pallas_call(
        functools.partial(_topk_body, n=n),
        grid=(b, m // bm),
        in_specs=[
            pl.BlockSpec((1, bm, 3), lambda i, j: (i, j, 0)),
            pl.BlockSpec((1, 3, n), lambda i, j: (i, 0, 0)),
        ],
        out_specs=pl.BlockSpec((1, bm, K), lambda i, j: (i, j, 0)),
        out_shape=jax.ShapeDtypeStruct((b, m, K), jnp.int32),
        scratch_shapes=[pltpu.VMEM((bm, n), jnp.float32)],
    )(qxyz_t, rxyz)


def _sc_gather(table_flat, idx_flat):
    g_total = idx_flat.shape[0]
    d = table_flat.shape[1]
    gpw = g_total // NW
    csz = min(128, gpw)
    nchunk = gpw // csz
    mesh = plsc.VectorSubcoreMesh(core_axis_name="c", subcore_axis_name="s")

    @functools.partial(
        pl.kernel,
        out_type=jax.ShapeDtypeStruct((g_total, d), jnp.float32),
        mesh=mesh,
        scratch_types=[
            pltpu.VMEM((gpw,), jnp.int32),
            pltpu.VMEM((csz, d), jnp.float32),
            pltpu.VMEM((csz, d), jnp.float32),
            pltpu.SemaphoreType.DMA,
            pltpu.SemaphoreType.DMA,
        ],
    )
    def gk(idx_hbm, tab_hbm, out_hbm, idx_v, buf0, buf1, semin, semout):
        wid = lax.axis_index("s") * NC + lax.axis_index("c")
        base = wid * gpw
        bufs = [buf0, buf1]
        pltpu.sync_copy(idx_hbm.at[pl.ds(base, gpw)], idx_v)
        hin = [None] * nchunk
        hout = [None] * nchunk
        hin[0] = pltpu.async_copy(
            tab_hbm.at[idx_v.at[pl.ds(0, csz)]], bufs[0], semin)
        for j in range(nchunk):
            if j + 1 < nchunk:
                if j >= 1:
                    hout[j - 1].wait()
                hin[j + 1] = pltpu.async_copy(
                    tab_hbm.at[idx_v.at[pl.ds((j + 1) * csz, csz)]],
                    bufs[(j + 1) % 2], semin)
            hin[j].wait()
            hout[j] = pltpu.async_copy(
                bufs[j % 2], out_hbm.at[pl.ds(base + j * csz, csz)], semout)
        hout[nchunk - 1].wait()
        if nchunk >= 2:
            hout[nchunk - 2].wait()

    return gk(idx_flat, table_flat)


def _stagec_body(g_ref, qx_ref, wn0_ref, wb0_ref, wn1_ref, wb1_ref, lw_ref,
                 lb_ref, o_ref, c, dpad):
    q = qx_ref[0]
    bmq = q.shape[0]

    fcol = jax.lax.broadcasted_iota(jnp.int32, (c, W2 * c), 1)
    frow = jax.lax.broadcasted_iota(jnp.int32, (c, W2 * c), 0)
    fmat = (fcol - (fcol // c) * c == frow).astype(jnp.float32)
    ecol = jax.lax.broadcasted_iota(jnp.int32, (W2, W2 * c), 1)
    erow = jax.lax.broadcasted_iota(jnp.int32, (W2, W2 * c), 0)
    emat = (ecol // c == erow).astype(jnp.float32)

    z = jnp.zeros((bmq, W2 * c), jnp.float32)
    for k in range(K):
        gk = g_ref[0, k]
        rel = gk[:, :3] - q
        h1 = jnp.maximum(
            jnp.dot(rel, wn0_ref[...], preferred_element_type=jnp.float32)
            + wb0_ref[...], 0.0)
        w = jnp.maximum(
            jnp.dot(h1, wn1_ref[...], preferred_element_type=jnp.float32)
            + wb1_ref[...], 0.0)
        f = gk[:, 3:3 + c]
        z = z + (jnp.dot(f, fmat, preferred_element_type=jnp.float32,
                         precision=jax.lax.Precision.HIGHEST)
                 * jnp.dot(w, emat, preferred_element_type=jnp.float32,
                           precision=jax.lax.Precision.HIGHEST))

    out = jnp.dot(z, lw_ref[...], preferred_element_type=jnp.float32) + lb_ref[...]
    o_ref[0] = jnp.maximum(out, 0.0)


def _stagec(g, qxyz_t, wn_w0, wn_b0, wn_w1, wn_b1, lin_w, lin_b, c, dpad, bmq):
    b, m, _ = qxyz_t.shape
    lw_perm = lin_w.reshape(c, c, W2).transpose(0, 2, 1).reshape(c, W2 * c).T
    return pl.pallas_call(
        functools.partial(_stagec_body, c=c, dpad=dpad),
        grid=(b, m // bmq),
        in_specs=[
            pl.BlockSpec((1, K, bmq, dpad), lambda i, j: (i, 0, j, 0)),
            pl.BlockSpec((1, bmq, 3), lambda i, j: (i, j, 0)),
            pl.BlockSpec((3, W1), lambda i, j: (0, 0)),
            pl.BlockSpec((1, W1), lambda i, j: (0, 0)),
            pl.BlockSpec((W1, W2), lambda i, j: (0, 0)),
            pl.BlockSpec((1, W2), lambda i, j: (0, 0)),
            pl.BlockSpec((W2 * c, c), lambda i, j: (0, 0)),
            pl.BlockSpec((1, c), lambda i, j: (0, 0)),
        ],
        out_specs=pl.BlockSpec((1, bmq, c), lambda i, j: (i, j, 0)),
        out_shape=jax.ShapeDtypeStruct((b, m, c), jnp.float32),
    )(g, qxyz_t, wn_w0.T, wn_b0.reshape(1, W1), wn_w1.T,
      wn_b1.reshape(1, W2), lw_perm, lin_b.reshape(1, -1))


def kernel(xyz0, xyz1, xyz2, xyz3,
           mlp0_w0, mlp0_b0, mlp0_w1, mlp0_b1,
           mlp1_w0, mlp1_b0, mlp1_w1, mlp1_b1,
           mlp2_w0, mlp2_b0, mlp2_w1, mlp2_b1,
           conv0_wn_w0, conv0_wn_b0, conv0_wn_w1, conv0_wn_b1, conv0_lin_w, conv0_lin_b,
           conv1_wn_w0, conv1_wn_b0, conv1_wn_w1, conv1_wn_b1, conv1_lin_w, conv1_lin_b,
           conv2_wn_w0, conv2_wn_b0, conv2_wn_w1, conv2_wn_b1, conv2_lin_w, conv2_lin_b):
    xyzs = [xyz0, xyz1, xyz2, xyz3]
    mlps = [(mlp0_w0, mlp0_b0, mlp0_w1, mlp0_b1),
            (mlp1_w0, mlp1_b0, mlp1_w1, mlp1_b1),
            (mlp2_w0, mlp2_b0, mlp2_w1, mlp2_b1)]
    convs = [(conv0_wn_w0, conv0_wn_b0, conv0_wn_w1, conv0_wn_b1, conv0_lin_w, conv0_lin_b),
             (conv1_wn_w0, conv1_wn_b0, conv1_wn_w1, conv1_wn_b1, conv1_lin_w, conv1_lin_b),
             (conv2_wn_w0, conv2_wn_b0, conv2_wn_w1, conv2_wn_b1, conv2_lin_w, conv2_lin_b)]
    dpads = [128, 128, 256]
    bmqs = [128, 64, 32]

    idxs = [_topk(xyzs[i], xyzs[i + 1].transpose(0, 2, 1), 128)
            for i in range(3)]

    outs = []
    prev = xyz0.transpose(0, 2, 1)
    for i in range(3):
        b, n, _ = prev.shape
        m = xyzs[i + 1].shape[2]
        w0, b0, w1, b1 = mlps[i]
        feat_t = _mlp(prev, w0, b0, w1, b1)
        c = feat_t.shape[-1]
        dpad = dpads[i]
        tab = jnp.concatenate(
            [xyzs[i].transpose(0, 2, 1), feat_t,
             jnp.zeros((b, n, dpad - 3 - c), jnp.float32)], axis=-1)
        idx_kmaj = idxs[i].transpose(0, 2, 1).reshape(-1)
        g = _sc_gather(tab.reshape(b * n, dpad), idx_kmaj)
        cwn0, cb0, cwn1, cb1, lw, lb = convs[i]
        qxyz_t = xyzs[i + 1].transpose(0, 2, 1)
        out_t = _stagec(g.reshape(b, K, m, dpad), qxyz_t, cwn0, cb0, cwn1,
                        cb1, lw, lb, c, dpad, bmqs[i])
        outs.append(out_t.transpose(0, 2, 1))
        prev = out_t
    return tuple(outs)


# speedup vs baseline: 1.4904x; 1.2710x over previous
import functools

import jax
import jax.numpy as jnp
from jax import lax
from jax.experimental import pallas as pl
from jax.experimental.pallas import tpu as pltpu
from jax.experimental.pallas import tpu_sc as plsc

K = 16
W1 = 10
W2 = 20
NC = 2
NS = 16
NW = NC * NS


def _mlp_body(x_ref, w0_ref, b0_ref, w1_ref, b1_ref, o_ref):
    x = x_ref[0]
    h = jnp.maximum(
        jnp.dot(x, w0_ref[...], preferred_element_type=jnp.float32) + b0_ref[...], 0.0)
    o = jnp.maximum(
        jnp.dot(h, w1_ref[...], preferred_element_type=jnp.float32) + b1_ref[...], 0.0)
    o_ref[0] = o


def _mlp(x, w0, b0, w1, b1):
    b, n, cin = x.shape
    h = w0.shape[0]
    cout = w1.shape[0]
    return pl.pallas_call(
        _mlp_body,
        grid=(b,),
        in_specs=[
            pl.BlockSpec((1, n, cin), lambda i: (i, 0, 0)),
            pl.BlockSpec((cin, h), lambda i: (0, 0)),
            pl.BlockSpec((1, h), lambda i: (0, 0)),
            pl.BlockSpec((h, cout), lambda i: (0, 0)),
            pl.BlockSpec((1, cout), lambda i: (0, 0)),
        ],
        out_specs=pl.BlockSpec((1, n, cout), lambda i: (i, 0, 0)),
        out_shape=jax.ShapeDtypeStruct((b, n, cout), jnp.float32),
    )(x, w0.T, b0.reshape(1, h), w1.T, b1.reshape(1, cout))


def _topk_body(qx_ref, rx_ref, o_ref, s_scr, n):
    bidx = pl.program_id(0)
    q = qx_ref[0]
    r3 = rx_ref[0]
    rsq = jnp.sum(r3 * r3, axis=0, keepdims=True)
    qsq = jnp.sum(q * q, axis=1, keepdims=True)
    s_scr[...] = (qsq + rsq) - 2.0 * jnp.dot(q, r3,
                                             preferred_element_type=jnp.float32)
    lane = jax.lax.broadcasted_iota(jnp.int32, s_scr.shape, 1)
    base = bidx * n
    for t in range(K):
        s = s_scr[...]
        am = jnp.argmin(s, axis=1, keepdims=True).astype(jnp.int32)
        s_scr[...] = jnp.where(lane == am, jnp.inf, s)
        o_ref[0, :, t:t + 1] = am + base


def _topk(rxyz, qxyz_t, bm):
    b, _, n = rxyz.shape
    m = qxyz_t.shape[1]
    return pl.pallas_call(
        functools.partial(_topk_body, n=n),
        grid=(b, m // bm),
        in_specs=[
            pl.BlockSpec((1, bm, 3), lambda i, j: (i, j, 0)),
            pl.BlockSpec((1, 3, n), lambda i, j: (i, 0, 0)),
        ],
        out_specs=pl.BlockSpec((1, bm, K), lambda i, j: (i, j, 0)),
        out_shape=jax.ShapeDtypeStruct((b, m, K), jnp.int32),
        scratch_shapes=[pltpu.VMEM((bm, n), jnp.float32)],
    )(qxyz_t, rxyz)


def _sc_gather(table_flat, idx_flat):
    g_total = idx_flat.shape[0]
    d = table_flat.shape[1]
    gpw = g_total // NW
    csz = min(128, gpw)
    nchunk = gpw // csz
    mesh = plsc.VectorSubcoreMesh(core_axis_name="c", subcore_axis_name="s")

    @functools.partial(
        pl.kernel,
        out_type=jax.ShapeDtypeStruct((g_total, d), jnp.float32),
        mesh=mesh,
        scratch_types=[
            pltpu.VMEM((gpw,), jnp.int32),
            pltpu.VMEM((csz, d), jnp.float32),
            pltpu.VMEM((csz, d), jnp.float32),
            pltpu.SemaphoreType.DMA,
            pltpu.SemaphoreType.DMA,
        ],
    )
    def gk(idx_hbm, tab_hbm, out_hbm, idx_v, buf0, buf1, semin, semout):
        wid = lax.axis_index("s") * NC + lax.axis_index("c")
        base = wid * gpw
        bufs = [buf0, buf1]
        pltpu.sync_copy(idx_hbm.at[pl.ds(base, gpw)], idx_v)
        hin = [None] * nchunk
        hout = [None] * nchunk
        hin[0] = pltpu.async_copy(
            tab_hbm.at[idx_v.at[pl.ds(0, csz)]], bufs[0], semin)
        for j in range(nchunk):
            if j + 1 < nchunk:
                if j >= 1:
                    hout[j - 1].wait()
                hin[j + 1] = pltpu.async_copy(
                    tab_hbm.at[idx_v.at[pl.ds((j + 1) * csz, csz)]],
                    bufs[(j + 1) % 2], semin)
            hin[j].wait()
            hout[j] = pltpu.async_copy(
                bufs[j % 2], out_hbm.at[pl.ds(base + j * csz, csz)], semout)
        hout[nchunk - 1].wait()
        if nchunk >= 2:
            hout[nchunk - 2].wait()

    return gk(idx_flat, table_flat)


def _stagec_body(g_ref, qx_ref, wn0_ref, wb0_ref, wn1_ref, wb1_ref, lw_ref,
                 lb_ref, o_ref, c, dpad):
    q = qx_ref[0]
    bmq = q.shape[0]
    gg = g_ref[0].reshape(K * bmq, dpad)
    qt = pltpu.repeat(q, K, axis=0)

    rel = gg[:, :3] - qt
    h1 = jnp.maximum(
        jnp.dot(rel, wn0_ref[...], preferred_element_type=jnp.float32)
        + wb0_ref[...], 0.0)
    w = jnp.maximum(
        jnp.dot(h1, wn1_ref[...], preferred_element_type=jnp.float32)
        + wb1_ref[...], 0.0)
    f = gg[:, 3:3 + c]

    ecol = jax.lax.broadcasted_iota(jnp.int32, (W2, W2 * c), 1)
    erow = jax.lax.broadcasted_iota(jnp.int32, (W2, W2 * c), 0)
    emat = (ecol // c == erow).astype(jnp.float32)
    we = jnp.dot(w, emat, preferred_element_type=jnp.float32,
                 precision=jax.lax.Precision.HIGHEST)
    fe = pltpu.repeat(f, W2, axis=1)
    p = (fe * we).reshape(K, bmq, W2 * c)
    z = jnp.sum(p, axis=0)

    out = jnp.dot(z, lw_ref[...], preferred_element_type=jnp.float32) + lb_ref[...]
    o_ref[0] = jnp.maximum(out, 0.0)


def _stagec(g, qxyz_t, wn_w0, wn_b0, wn_w1, wn_b1, lin_w, lin_b, c, dpad, bmq):
    b, m, _ = qxyz_t.shape
    lw_perm = lin_w.reshape(c, c, W2).transpose(0, 2, 1).reshape(c, W2 * c).T
    return pl.pallas_call(
        functools.partial(_stagec_body, c=c, dpad=dpad),
        grid=(b, m // bmq),
        in_specs=[
            pl.BlockSpec((1, K, bmq, dpad), lambda i, j: (i, 0, j, 0)),
            pl.BlockSpec((1, bmq, 3), lambda i, j: (i, j, 0)),
            pl.BlockSpec((3, W1), lambda i, j: (0, 0)),
            pl.BlockSpec((1, W1), lambda i, j: (0, 0)),
            pl.BlockSpec((W1, W2), lambda i, j: (0, 0)),
            pl.BlockSpec((1, W2), lambda i, j: (0, 0)),
            pl.BlockSpec((W2 * c, c), lambda i, j: (0, 0)),
            pl.BlockSpec((1, c), lambda i, j: (0, 0)),
        ],
        out_specs=pl.BlockSpec((1, bmq, c), lambda i, j: (i, j, 0)),
        out_shape=jax.ShapeDtypeStruct((b, m, c), jnp.float32),
    )(g, qxyz_t, wn_w0.T, wn_b0.reshape(1, W1), wn_w1.T,
      wn_b1.reshape(1, W2), lw_perm, lin_b.reshape(1, -1))


def kernel(xyz0, xyz1, xyz2, xyz3,
           mlp0_w0, mlp0_b0, mlp0_w1, mlp0_b1,
           mlp1_w0, mlp1_b0, mlp1_w1, mlp1_b1,
           mlp2_w0, mlp2_b0, mlp2_w1, mlp2_b1,
           conv0_wn_w0, conv0_wn_b0, conv0_wn_w1, conv0_wn_b1, conv0_lin_w, conv0_lin_b,
           conv1_wn_w0, conv1_wn_b0, conv1_wn_w1, conv1_wn_b1, conv1_lin_w, conv1_lin_b,
           conv2_wn_w0, conv2_wn_b0, conv2_wn_w1, conv2_wn_b1, conv2_lin_w, conv2_lin_b):
    xyzs = [xyz0, xyz1, xyz2, xyz3]
    mlps = [(mlp0_w0, mlp0_b0, mlp0_w1, mlp0_b1),
            (mlp1_w0, mlp1_b0, mlp1_w1, mlp1_b1),
            (mlp2_w0, mlp2_b0, mlp2_w1, mlp2_b1)]
    convs = [(conv0_wn_w0, conv0_wn_b0, conv0_wn_w1, conv0_wn_b1, conv0_lin_w, conv0_lin_b),
             (conv1_wn_w0, conv1_wn_b0, conv1_wn_w1, conv1_wn_b1, conv1_lin_w, conv1_lin_b),
             (conv2_wn_w0, conv2_wn_b0, conv2_wn_w1, conv2_wn_b1, conv2_lin_w, conv2_lin_b)]
    dpads = [128, 128, 256]
    bmqs = [128, 64, 32]

    idxs = [_topk(xyzs[i], xyzs[i + 1].transpose(0, 2, 1), 128)
            for i in range(3)]

    outs = []
    prev = xyz0.transpose(0, 2, 1)
    for i in range(3):
        b, n, _ = prev.shape
        m = xyzs[i + 1].shape[2]
        w0, b0, w1, b1 = mlps[i]
        feat_t = _mlp(prev, w0, b0, w1, b1)
        c = feat_t.shape[-1]
        dpad = dpads[i]
        tab = jnp.concatenate(
            [xyzs[i].transpose(0, 2, 1), feat_t,
             jnp.zeros((b, n, dpad - 3 - c), jnp.float32)], axis=-1)
        idx_kmaj = idxs[i].transpose(0, 2, 1).reshape(-1)
        g = _sc_gather(tab.reshape(b * n, dpad), idx_kmaj)
        cwn0, cb0, cwn1, cb1, lw, lb = convs[i]
        qxyz_t = xyzs[i + 1].transpose(0, 2, 1)
        out_t = _stagec(g.reshape(b, K, m, dpad), qxyz_t, cwn0, cb0, cwn1,
                        cb1, lw, lb, c, dpad, bmqs[i])
        outs.append(out_t.transpose(0, 2, 1))
        prev = out_t
    return tuple(outs)


# speedup vs baseline: 1.8770x; 1.2594x over previous
import functools

import jax
import jax.numpy as jnp
from jax import lax
from jax.experimental import pallas as pl
from jax.experimental.pallas import tpu as pltpu
from jax.experimental.pallas import tpu_sc as plsc

K = 16
W1 = 10
W2 = 20
NC = 2
NS = 16
NW = NC * NS


def _mlp_body(x_ref, w0_ref, b0_ref, w1_ref, b1_ref, o_ref):
    x = x_ref[0]
    h = jnp.maximum(
        jnp.dot(x, w0_ref[...], preferred_element_type=jnp.float32) + b0_ref[...], 0.0)
    o = jnp.maximum(
        jnp.dot(h, w1_ref[...], preferred_element_type=jnp.float32) + b1_ref[...], 0.0)
    o_ref[0] = o


def _mlp(x, w0, b0, w1, b1):
    b, n, cin = x.shape
    h = w0.shape[0]
    cout = w1.shape[0]
    return pl.pallas_call(
        _mlp_body,
        grid=(b,),
        in_specs=[
            pl.BlockSpec((1, n, cin), lambda i: (i, 0, 0)),
            pl.BlockSpec((cin, h), lambda i: (0, 0)),
            pl.BlockSpec((1, h), lambda i: (0, 0)),
            pl.BlockSpec((h, cout), lambda i: (0, 0)),
            pl.BlockSpec((1, cout), lambda i: (0, 0)),
        ],
        out_specs=pl.BlockSpec((1, n, cout), lambda i: (i, 0, 0)),
        out_shape=jax.ShapeDtypeStruct((b, n, cout), jnp.float32),
    )(x, w0.T, b0.reshape(1, h), w1.T, b1.reshape(1, cout))


def _topk_body(qx_ref, rx_ref, o_ref, s_scr, n):
    bidx = pl.program_id(0)
    q = qx_ref[0]
    r3 = rx_ref[0]
    rsq = jnp.sum(r3 * r3, axis=0, keepdims=True)
    qsq = jnp.sum(q * q, axis=1, keepdims=True)
    s_scr[...] = (qsq + rsq) - 2.0 * jnp.dot(q, r3,
                                             preferred_element_type=jnp.float32)
    lane = jax.lax.broadcasted_iota(jnp.int32, s_scr.shape, 1)
    base = bidx * n
    for t in range(K):
        s = s_scr[...]
        am = jnp.argmin(s, axis=1, keepdims=True).astype(jnp.int32)
        s_scr[...] = jnp.where(lane == am, jnp.inf, s)
        o_ref[0, :, t:t + 1] = am + base


def _topk(rxyz, qxyz_t, bm):
    b, _, n = rxyz.shape
    m = qxyz_t.shape[1]
    return pl.pallas_call(
        functools.partial(_topk_body, n=n),
        grid=(b, m // bm),
        in_specs=[
            pl.BlockSpec((1, bm, 3), lambda i, j: (i, j, 0)),
            pl.BlockSpec((1, 3, n), lambda i, j: (i, 0, 0)),
        ],
        out_specs=pl.BlockSpec((1, bm, K), lambda i, j: (i, j, 0)),
        out_shape=jax.ShapeDtypeStruct((b, m, K), jnp.int32),
        scratch_shapes=[pltpu.VMEM((bm, n), jnp.float32)],
    )(qxyz_t, rxyz)


def _sc_gather(table_flat, idx_flat):
    g_total = idx_flat.shape[0]
    d = table_flat.shape[1]
    gpw = g_total // NW
    csz = min(128, gpw)
    nchunk = gpw // csz
    mesh = plsc.VectorSubcoreMesh(core_axis_name="c", subcore_axis_name="s")

    @functools.partial(
        pl.kernel,
        out_type=jax.ShapeDtypeStruct((g_total, d), jnp.float32),
        mesh=mesh,
        scratch_types=[
            pltpu.VMEM((gpw,), jnp.int32),
            pltpu.VMEM((csz, d), jnp.float32),
            pltpu.VMEM((csz, d), jnp.float32),
            pltpu.SemaphoreType.DMA,
            pltpu.SemaphoreType.DMA,
        ],
    )
    def gk(idx_hbm, tab_hbm, out_hbm, idx_v, buf0, buf1, semin, semout):
        wid = lax.axis_index("s") * NC + lax.axis_index("c")
        base = wid * gpw
        bufs = [buf0, buf1]
        pltpu.sync_copy(idx_hbm.at[pl.ds(base, gpw)], idx_v)
        hin = [None] * nchunk
        hout = [None] * nchunk
        hin[0] = pltpu.async_copy(
            tab_hbm.at[idx_v.at[pl.ds(0, csz)]], bufs[0], semin)
        for j in range(nchunk):
            if j + 1 < nchunk:
                if j >= 1:
                    hout[j - 1].wait()
                hin[j + 1] = pltpu.async_copy(
                    tab_hbm.at[idx_v.at[pl.ds((j + 1) * csz, csz)]],
                    bufs[(j + 1) % 2], semin)
            hin[j].wait()
            hout[j] = pltpu.async_copy(
                bufs[j % 2], out_hbm.at[pl.ds(base + j * csz, csz)], semout)
        hout[nchunk - 1].wait()
        if nchunk >= 2:
            hout[nchunk - 2].wait()

    return gk(idx_flat, table_flat)


def _stagec_body(g_ref, qx_ref, wn0_ref, wb0_ref, wn1_ref, wb1_ref, lw_ref,
                 lb_ref, o_ref, c, dpad):
    q = qx_ref[0]
    bmq = q.shape[0]
    gg = g_ref[0].reshape(K * bmq, dpad)
    qt = pltpu.repeat(q, K, axis=0)

    rel = gg[:, :3] - qt
    h1 = jnp.maximum(
        jnp.dot(rel, wn0_ref[...], preferred_element_type=jnp.float32)
        + wb0_ref[...], 0.0)
    w = jnp.maximum(
        jnp.dot(h1, wn1_ref[...], preferred_element_type=jnp.float32)
        + wb1_ref[...], 0.0)
    f = gg[:, 3:3 + c]

    ecol = jax.lax.broadcasted_iota(jnp.int32, (W2, W2 * c), 1)
    erow = jax.lax.broadcasted_iota(jnp.int32, (W2, W2 * c), 0)
    emat = (ecol // c == erow).astype(jnp.float32)
    we = jnp.dot(w, emat, preferred_element_type=jnp.float32)
    fe = pltpu.repeat(f, W2, axis=1)
    p = (fe * we).reshape(K, bmq, W2 * c)
    z = jnp.sum(p, axis=0)

    out = jnp.dot(z, lw_ref[...], preferred_element_type=jnp.float32) + lb_ref[...]
    o_ref[0] = jnp.maximum(out, 0.0)


def _stagec(g, qxyz_t, wn_w0, wn_b0, wn_w1, wn_b1, lin_w, lin_b, c, dpad, bmq):
    b, m, _ = qxyz_t.shape
    lw_perm = lin_w.reshape(c, c, W2).transpose(0, 2, 1).reshape(c, W2 * c).T
    return pl.pallas_call(
        functools.partial(_stagec_body, c=c, dpad=dpad),
        grid=(b, m // bmq),
        in_specs=[
            pl.BlockSpec((1, K, bmq, dpad), lambda i, j: (i, 0, j, 0)),
            pl.BlockSpec((1, bmq, 3), lambda i, j: (i, j, 0)),
            pl.BlockSpec((3, W1), lambda i, j: (0, 0)),
            pl.BlockSpec((1, W1), lambda i, j: (0, 0)),
            pl.BlockSpec((W1, W2), lambda i, j: (0, 0)),
            pl.BlockSpec((1, W2), lambda i, j: (0, 0)),
            pl.BlockSpec((W2 * c, c), lambda i, j: (0, 0)),
            pl.BlockSpec((1, c), lambda i, j: (0, 0)),
        ],
        out_specs=pl.BlockSpec((1, bmq, c), lambda i, j: (i, j, 0)),
        out_shape=jax.ShapeDtypeStruct((b, m, c), jnp.float32),
    )(g, qxyz_t, wn_w0.T, wn_b0.reshape(1, W1), wn_w1.T,
      wn_b1.reshape(1, W2), lw_perm, lin_b.reshape(1, -1))


def kernel(xyz0, xyz1, xyz2, xyz3,
           mlp0_w0, mlp0_b0, mlp0_w1, mlp0_b1,
           mlp1_w0, mlp1_b0, mlp1_w1, mlp1_b1,
           mlp2_w0, mlp2_b0, mlp2_w1, mlp2_b1,
           conv0_wn_w0, conv0_wn_b0, conv0_wn_w1, conv0_wn_b1, conv0_lin_w, conv0_lin_b,
           conv1_wn_w0, conv1_wn_b0, conv1_wn_w1, conv1_wn_b1, conv1_lin_w, conv1_lin_b,
           conv2_wn_w0, conv2_wn_b0, conv2_wn_w1, conv2_wn_b1, conv2_lin_w, conv2_lin_b):
    xyzs = [xyz0, xyz1, xyz2, xyz3]
    mlps = [(mlp0_w0, mlp0_b0, mlp0_w1, mlp0_b1),
            (mlp1_w0, mlp1_b0, mlp1_w1, mlp1_b1),
            (mlp2_w0, mlp2_b0, mlp2_w1, mlp2_b1)]
    convs = [(conv0_wn_w0, conv0_wn_b0, conv0_wn_w1, conv0_wn_b1, conv0_lin_w, conv0_lin_b),
             (conv1_wn_w0, conv1_wn_b0, conv1_wn_w1, conv1_wn_b1, conv1_lin_w, conv1_lin_b),
             (conv2_wn_w0, conv2_wn_b0, conv2_wn_w1, conv2_wn_b1, conv2_lin_w, conv2_lin_b)]
    dpads = [128, 128, 256]
    bmqs = [128, 64, 32]

    idxs = [_topk(xyzs[i], xyzs[i + 1].transpose(0, 2, 1), 128)
            for i in range(3)]

    outs = []
    prev = xyz0.transpose(0, 2, 1)
    for i in range(3):
        b, n, _ = prev.shape
        m = xyzs[i + 1].shape[2]
        w0, b0, w1, b1 = mlps[i]
        feat_t = _mlp(prev, w0, b0, w1, b1)
        c = feat_t.shape[-1]
        dpad = dpads[i]
        tab = jnp.concatenate(
            [xyzs[i].transpose(0, 2, 1), feat_t,
             jnp.zeros((b, n, dpad - 3 - c), jnp.float32)], axis=-1)
        idx_kmaj = idxs[i].transpose(0, 2, 1).reshape(-1)
        g = _sc_gather(tab.reshape(b * n, dpad), idx_kmaj)
        cwn0, cb0, cwn1, cb1, lw, lb = convs[i]
        qxyz_t = xyzs[i + 1].transpose(0, 2, 1)
        out_t = _stagec(g.reshape(b, K, m, dpad), qxyz_t, cwn0, cb0, cwn1,
                        cb1, lw, lb, c, dpad, bmqs[i])
        outs.append(out_t.transpose(0, 2, 1))
        prev = out_t
    return tuple(outs)
